# baseline (speedup 1.0000x reference)
import functools

import jax
import jax.numpy as jnp
from jax import lax
from jax.experimental import pallas as pl
from jax.experimental.pallas import tpu as pltpu
from jax.experimental.pallas import tpu_sc as plsc

N = 10000
E = 320000
D_FEAT = 128
D_EDGE = 16
H = 64
L = 3
EPS = 1e-7

_NC = 2
_NS = 16
_NW = _NC * _NS
_EW = E // _NW
_C = 40
_K = _EW // _C
_NPAD = 10240
_RPT = _NPAD // _NS

_f32 = jnp.float32


def _node_enc(x, w, b):
    R = 2000

    def body(x_ref, w_ref, b_ref, h_ref, hmax_ref):
        h = jnp.dot(x_ref[...], w_ref[...], preferred_element_type=_f32) + b_ref[...]
        h_ref[...] = h
        m8 = jnp.broadcast_to(jnp.max(h, axis=0, keepdims=True), (8, H))

        @pl.when(pl.program_id(0) == 0)
        def _():
            hmax_ref[...] = m8

        @pl.when(pl.program_id(0) > 0)
        def _():
            hmax_ref[...] = jnp.maximum(hmax_ref[...], m8)

    return pl.pallas_call(
        body,
        grid=(N // R,),
        in_specs=[
            pl.BlockSpec((R, D_FEAT), lambda i: (i, 0)),
            pl.BlockSpec((D_FEAT, H), lambda i: (0, 0)),
            pl.BlockSpec((1, H), lambda i: (0, 0)),
        ],
        out_specs=[
            pl.BlockSpec((R, H), lambda i: (i, 0)),
            pl.BlockSpec((8, H), lambda i: (0, 0)),
        ],
        out_shape=[
            jax.ShapeDtypeStruct((N, H), _f32),
            jax.ShapeDtypeStruct((8, H), _f32),
        ],
    )(x, w, b)


def _edge_mlp(a, w1, b1, w2, b2):
    R = 8000

    def body(a_ref, w1_ref, b1_ref, w2_ref, b2_ref, ea_ref, emax_ref):
        u = jnp.maximum(
            jnp.dot(a_ref[...], w1_ref[...], preferred_element_type=_f32) + b1_ref[...], 0.0)
        ea = jnp.dot(u, w2_ref[...], preferred_element_type=_f32) + b2_ref[...]
        ea_ref[...] = ea
        m8 = jnp.broadcast_to(jnp.max(ea, axis=0, keepdims=True), (8, H))

        @pl.when(pl.program_id(0) == 0)
        def _():
            emax_ref[...] = m8

        @pl.when(pl.program_id(0) > 0)
        def _():
            emax_ref[...] = jnp.maximum(emax_ref[...], m8)

    return pl.pallas_call(
        body,
        grid=(E // R,),
        in_specs=[
            pl.BlockSpec((R, D_EDGE), lambda i: (i, 0)),
            pl.BlockSpec((D_EDGE, 32), lambda i: (0, 0)),
            pl.BlockSpec((1, 32), lambda i: (0, 0)),
            pl.BlockSpec((32, H), lambda i: (0, 0)),
            pl.BlockSpec((1, H), lambda i: (0, 0)),
        ],
        out_specs=[
            pl.BlockSpec((R, H), lambda i: (i, 0)),
            pl.BlockSpec((8, H), lambda i: (0, 0)),
        ],
        out_shape=[
            jax.ShapeDtypeStruct((E, H), _f32),
            jax.ShapeDtypeStruct((8, H), _f32),
        ],
    )(a, w1, b1, w2, b2)


def _combine(parts, hin, res, w1, b1, g1, bb1, w2, b2, lng, lnb):
    R = 2000

    def body(p0_ref, p1_ref, hin_ref, res_ref, w1_ref, b1_ref, g1_ref, bb1_ref,
             w2_ref, b2_ref, lng_ref, lnb_ref, h_ref, hn_ref, hmax_ref):
        p0 = p0_ref[0]
        p1 = p1_ref[0]
        num = p0[:, :H] + p1[:, :H]
        den = p0[:, H:] + p1[:, H:]
        aggr = num / (den + 1e-16)
        v = aggr + hin_ref[...]
        u = jnp.dot(v, w1_ref[...], preferred_element_type=_f32) + b1_ref[...]
        mu = jnp.mean(u, axis=-1, keepdims=True)
        var = jnp.mean((u - mu) ** 2, axis=-1, keepdims=True)
        u = (u - mu) / jnp.sqrt(var + 1e-5) * g1_ref[...] + bb1_ref[...]
        u = jnp.maximum(u, 0.0)
        h = res_ref[...] + jnp.dot(u, w2_ref[...], preferred_element_type=_f32) + b2_ref[...]
        h_ref[...] = h
        mu2 = jnp.mean(h, axis=-1, keepdims=True)
        var2 = jnp.mean((h - mu2) ** 2, axis=-1, keepdims=True)
        hn = jnp.maximum((h - mu2) / jnp.sqrt(var2 + 1e-5) * lng_ref[...] + lnb_ref[...], 0.0)
        hn_ref[...] = hn
        m8 = jnp.broadcast_to(jnp.max(hn, axis=0, keepdims=True), (8, H))

        @pl.when(pl.program_id(0) == 0)
        def _():
            hmax_ref[...] = m8

        @pl.when(pl.program_id(0) > 0)
        def _():
            hmax_ref[...] = jnp.maximum(hmax_ref[...], m8)

    nblk = N // R
    return pl.pallas_call(
        body,
        grid=(nblk,),
        in_specs=[
            pl.BlockSpec((1, R, 2 * H), lambda i: (0, i, 0)),
            pl.BlockSpec((1, R, 2 * H), lambda i: (1, i, 0)),
            pl.BlockSpec((R, H), lambda i: (i, 0)),
            pl.BlockSpec((R, H), lambda i: (i, 0)),
            pl.BlockSpec((H, 2 * H), lambda i: (0, 0)),
            pl.BlockSpec((1, 2 * H), lambda i: (0, 0)),
            pl.BlockSpec((1, 2 * H), lambda i: (0, 0)),
            pl.BlockSpec((1, 2 * H), lambda i: (0, 0)),
            pl.BlockSpec((2 * H, H), lambda i: (0, 0)),
            pl.BlockSpec((1, H), lambda i: (0, 0)),
            pl.BlockSpec((1, H), lambda i: (0, 0)),
            pl.BlockSpec((1, H), lambda i: (0, 0)),
        ],
        out_specs=[
            pl.BlockSpec((R, H), lambda i: (i, 0)),
            pl.BlockSpec((R, H), lambda i: (i, 0)),
            pl.BlockSpec((8, H), lambda i: (0, 0)),
        ],
        out_shape=[
            jax.ShapeDtypeStruct((N, H), _f32),
            jax.ShapeDtypeStruct((N, H), _f32),
            jax.ShapeDtypeStruct((8, H), _f32),
        ],
    )(parts, parts, hin, res, w1, b1, g1, bb1, w2, b2, lng, lnb)


def _edge_pass(h, ea, srcr, dstr, par, zrows):
    mesh = plsc.VectorSubcoreMesh(core_axis_name="c", subcore_axis_name="s")

    @functools.partial(
        pl.kernel,
        out_type=jax.ShapeDtypeStruct((_NC, _NPAD, 2 * H), _f32),
        mesh=mesh,
        scratch_types=[
            pltpu.VMEM((_K, _C), jnp.int32),
            pltpu.VMEM((_K, _C), jnp.int32),
            pltpu.VMEM((2, _C, H), _f32),
            pltpu.VMEM((2, _C, H), _f32),
            pltpu.VMEM((2, _C, 2 * H), _f32),
            pltpu.VMEM((2 * H,), _f32),
            pltpu.VMEM_SHARED((_NPAD, 2 * H), _f32),
            pltpu.SemaphoreType.DMA,
            pltpu.SemaphoreType.DMA,
            pltpu.SemaphoreType.DMA,
            pltpu.SemaphoreType.DMA,
            pltpu.SemaphoreType.DMA,
            pltpu.SemaphoreType.DMA,
        ],
        compiler_params=pltpu.CompilerParams(use_tc_tiling_on_sc=False),
    )
    def kern(h_hbm, ea_hbm, src_hbm, dst_hbm, par_hbm, z_hbm, out_hbm,
             sidx, didx, gbuf, ebuf, obuf, pbuf, acc,
             semg0, semg1, seme0, seme1, sems0, sems1):
        cid = lax.axis_index("c")
        sid = lax.axis_index("s")
        wid = cid * _NS + sid
        semg = (semg0, semg1)
        seme = (seme0, seme1)
        sems = (sems0, sems1)

        pltpu.sync_copy(par_hbm, pbuf)
        pltpu.sync_copy(src_hbm.at[pl.ds(wid * _K, _K)], sidx)
        pltpu.sync_copy(dst_hbm.at[pl.ds(wid * _K, _K)], didx)
        pltpu.sync_copy(z_hbm, acc.at[pl.ds(sid * _RPT, _RPT)])
        plsc.subcore_barrier()

        tv = pbuf[pl.ds(H, 16)]
        dvs = tuple(pbuf[pl.ds(16 * q, 16)] for q in range(4))

        def fire(k, b):
            pltpu.async_copy(h_hbm.at[sidx.at[k]], gbuf.at[b], semg[b])
            base = wid * _EW + k * _C
            pltpu.async_copy(ea_hbm.at[pl.ds(base, _C)], ebuf.at[b], seme[b])

        def wait_in(b):
            pltpu.make_async_copy(h_hbm.at[sidx.at[0]], gbuf.at[b], semg[b]).wait()
            pltpu.make_async_copy(ea_hbm.at[pl.ds(0, _C)], ebuf.at[b], seme[b]).wait()

        def compute(b):
            @plsc.parallel_loop(0, _C, 1, unroll=4)
            def row(i):
                gs = [gbuf[b, i, pl.ds(16 * q, 16)] for q in range(4)]
                es = [ebuf[b, i, pl.ds(16 * q, 16)] for q in range(4)]
                vs = [jnp.maximum(gs[q] + es[q], 0.0) for q in range(4)]
                ws = [jnp.exp(tv * vs[q] + dvs[q]) for q in range(4)]
                for q in range(4):
                    obuf[b, i, pl.ds(16 * q, 16)] = (vs[q] + EPS) * ws[q]
                    obuf[b, i, pl.ds(H + 16 * q, 16)] = ws[q]

        def scat(k, b):
            pltpu.async_copy(obuf.at[b], acc.at[didx.at[k]], sems[b], add=True)

        def wait_scat(b):
            pltpu.make_async_copy(obuf.at[b], acc.at[didx.at[0]], sems[b]).wait()

        fire(0, 0)
        fire(1, 1)

        def pair(p, carry):
            k0 = 2 * p
            wait_in(0)

            @pl.when(p >= 1)
            def _():
                wait_scat(0)

            compute(0)
            scat(k0, 0)

            @pl.when(k0 + 2 < _K)
            def _():
                fire(k0 + 2, 0)

            wait_in(1)

            @pl.when(p >= 1)
            def _():
                wait_scat(1)

            compute(1)
            scat(k0 + 1, 1)

            @pl.when(k0 + 3 < _K)
            def _():
                fire(k0 + 3, 1)

            return carry

        lax.fori_loop(0, _K // 2, pair, 0)

        wait_scat(0)
        wait_scat(1)

        plsc.subcore_barrier()
        pltpu.sync_copy(acc.at[pl.ds(sid * _RPT, _RPT)],
                        out_hbm.at[cid, pl.ds(sid * _RPT, _RPT)])

    return kern(h, ea, srcr, dstr, par, zrows)


def kernel(x, edge_index, edge_attr, batch, node_W, node_b, e1_W, e1_b, e2_W,
           e2_b, ln_g, ln_b, mlp1_W, mlp1_b, mln_g, mln_b, mlp2_W, mlp2_b, t,
           out_W, out_b):
    srcr = edge_index[0].reshape(E // _C, _C)
    dstr = edge_index[1].reshape(E // _C, _C)

    h0, hmax8 = _node_enc(x, node_W, node_b.reshape(1, H))
    ea, emax8 = _edge_mlp(edge_attr, e1_W, e1_b.reshape(1, 32), e2_W,
                          e2_b.reshape(1, H))
    eamax = emax8[0]

    zrows = jnp.zeros((_RPT, 2 * H), _f32)
    res = jnp.zeros((N, H), _f32)
    hin = h0
    hmaxv = hmax8[0]

    for i in range(L):
        ti = t[i]
        c = jnp.maximum(ti * (jnp.maximum(hmaxv + eamax, 0.0) + EPS), ti * EPS)
        par = jnp.concatenate([ti * EPS - c, jnp.full((H,), 1.0, _f32) * ti])
        parts = _edge_pass(hin, ea, srcr, dstr, par, zrows)
        if i < L - 1:
            lng, lnb = ln_g[i + 1], ln_b[i + 1]
        else:
            lng, lnb = ln_g[0], ln_b[0]
        h_new, hn, hmax8 = _combine(
            parts, hin, res,
            mlp1_W[i], mlp1_b[i].reshape(1, 2 * H),
            mln_g[i].reshape(1, 2 * H), mln_b[i].reshape(1, 2 * H),
            mlp2_W[i], mlp2_b[i].reshape(1, H),
            lng.reshape(1, H), lnb.reshape(1, H))
        res = h_new
        hin = hn
        hmaxv = hmax8[0]

    pooled = hmaxv.reshape(1, H)
    return jax.nn.sigmoid(pooled @ out_W + out_b)


# speedup vs baseline: 1.0920x; 1.0920x over previous
import functools

import jax
import jax.numpy as jnp
from jax import lax
from jax.experimental import pallas as pl
from jax.experimental.pallas import tpu as pltpu
from jax.experimental.pallas import tpu_sc as plsc

N = 10000
E = 320000
D_FEAT = 128
D_EDGE = 16
H = 64
L = 3
EPS = 1e-7

_NC = 2
_NS = 16
_NW = _NC * _NS
_EW = E // _NW
_C = 40
_K = _EW // _C
_NPAD = 10240
_RPT = _NPAD // _NS

_f32 = jnp.float32


def _node_enc(x, w, b):
    R = 2000

    def body(x_ref, w_ref, b_ref, h_ref, hmax_ref):
        h = jnp.dot(x_ref[...], w_ref[...], preferred_element_type=_f32) + b_ref[...]
        h_ref[...] = h
        m8 = jnp.broadcast_to(jnp.max(h, axis=0, keepdims=True), (8, H))

        @pl.when(pl.program_id(0) == 0)
        def _():
            hmax_ref[...] = m8

        @pl.when(pl.program_id(0) > 0)
        def _():
            hmax_ref[...] = jnp.maximum(hmax_ref[...], m8)

    return pl.pallas_call(
        body,
        grid=(N // R,),
        in_specs=[
            pl.BlockSpec((R, D_FEAT), lambda i: (i, 0)),
            pl.BlockSpec((D_FEAT, H), lambda i: (0, 0)),
            pl.BlockSpec((1, H), lambda i: (0, 0)),
        ],
        out_specs=[
            pl.BlockSpec((R, H), lambda i: (i, 0)),
            pl.BlockSpec((8, H), lambda i: (0, 0)),
        ],
        out_shape=[
            jax.ShapeDtypeStruct((N, H), _f32),
            jax.ShapeDtypeStruct((8, H), _f32),
        ],
    )(x, w, b)


def _edge_mlp(a, w1, b1, w2, b2):
    R = 8000

    def body(a_ref, w1_ref, b1_ref, w2_ref, b2_ref, ea_ref, emax_ref):
        u = jnp.maximum(
            jnp.dot(a_ref[...], w1_ref[...], preferred_element_type=_f32) + b1_ref[...], 0.0)
        ea = jnp.dot(u, w2_ref[...], preferred_element_type=_f32) + b2_ref[...]
        ea_ref[...] = ea
        m8 = jnp.broadcast_to(jnp.max(ea, axis=0, keepdims=True), (8, H))

        @pl.when(pl.program_id(0) == 0)
        def _():
            emax_ref[...] = m8

        @pl.when(pl.program_id(0) > 0)
        def _():
            emax_ref[...] = jnp.maximum(emax_ref[...], m8)

    return pl.pallas_call(
        body,
        grid=(E // R,),
        in_specs=[
            pl.BlockSpec((R, D_EDGE), lambda i: (i, 0)),
            pl.BlockSpec((D_EDGE, 32), lambda i: (0, 0)),
            pl.BlockSpec((1, 32), lambda i: (0, 0)),
            pl.BlockSpec((32, H), lambda i: (0, 0)),
            pl.BlockSpec((1, H), lambda i: (0, 0)),
        ],
        out_specs=[
            pl.BlockSpec((R, H), lambda i: (i, 0)),
            pl.BlockSpec((8, H), lambda i: (0, 0)),
        ],
        out_shape=[
            jax.ShapeDtypeStruct((E, H), _f32),
            jax.ShapeDtypeStruct((8, H), _f32),
        ],
    )(a, w1, b1, w2, b2)


def _combine(parts, hin, res, w1, b1, g1, bb1, w2, b2, lng, lnb):
    R = 2000

    def body(p0_ref, p1_ref, hin_ref, res_ref, w1_ref, b1_ref, g1_ref, bb1_ref,
             w2_ref, b2_ref, lng_ref, lnb_ref, h_ref, hn_ref, hmax_ref):
        p0 = p0_ref[0]
        p1 = p1_ref[0]
        num = p0[:, :H] + p1[:, :H]
        den = p0[:, H:] + p1[:, H:]
        aggr = num / (den + 1e-16)
        v = aggr + hin_ref[...]
        u = jnp.dot(v, w1_ref[...], preferred_element_type=_f32) + b1_ref[...]
        mu = jnp.mean(u, axis=-1, keepdims=True)
        var = jnp.mean((u - mu) ** 2, axis=-1, keepdims=True)
        u = (u - mu) / jnp.sqrt(var + 1e-5) * g1_ref[...] + bb1_ref[...]
        u = jnp.maximum(u, 0.0)
        h = res_ref[...] + jnp.dot(u, w2_ref[...], preferred_element_type=_f32) + b2_ref[...]
        h_ref[...] = h
        mu2 = jnp.mean(h, axis=-1, keepdims=True)
        var2 = jnp.mean((h - mu2) ** 2, axis=-1, keepdims=True)
        hn = jnp.maximum((h - mu2) / jnp.sqrt(var2 + 1e-5) * lng_ref[...] + lnb_ref[...], 0.0)
        hn_ref[...] = hn
        m8 = jnp.broadcast_to(jnp.max(hn, axis=0, keepdims=True), (8, H))

        @pl.when(pl.program_id(0) == 0)
        def _():
            hmax_ref[...] = m8

        @pl.when(pl.program_id(0) > 0)
        def _():
            hmax_ref[...] = jnp.maximum(hmax_ref[...], m8)

    nblk = N // R
    return pl.pallas_call(
        body,
        grid=(nblk,),
        in_specs=[
            pl.BlockSpec((1, R, 2 * H), lambda i: (0, i, 0)),
            pl.BlockSpec((1, R, 2 * H), lambda i: (1, i, 0)),
            pl.BlockSpec((R, H), lambda i: (i, 0)),
            pl.BlockSpec((R, H), lambda i: (i, 0)),
            pl.BlockSpec((H, 2 * H), lambda i: (0, 0)),
            pl.BlockSpec((1, 2 * H), lambda i: (0, 0)),
            pl.BlockSpec((1, 2 * H), lambda i: (0, 0)),
            pl.BlockSpec((1, 2 * H), lambda i: (0, 0)),
            pl.BlockSpec((2 * H, H), lambda i: (0, 0)),
            pl.BlockSpec((1, H), lambda i: (0, 0)),
            pl.BlockSpec((1, H), lambda i: (0, 0)),
            pl.BlockSpec((1, H), lambda i: (0, 0)),
        ],
        out_specs=[
            pl.BlockSpec((R, H), lambda i: (i, 0)),
            pl.BlockSpec((R, H), lambda i: (i, 0)),
            pl.BlockSpec((8, H), lambda i: (0, 0)),
        ],
        out_shape=[
            jax.ShapeDtypeStruct((N, H), _f32),
            jax.ShapeDtypeStruct((N, H), _f32),
            jax.ShapeDtypeStruct((8, H), _f32),
        ],
    )(parts, parts, hin, res, w1, b1, g1, bb1, w2, b2, lng, lnb)


def _edge_pass(h, ea, srcr, dstr, par, zrows):
    mesh = plsc.VectorSubcoreMesh(core_axis_name="c", subcore_axis_name="s")

    @functools.partial(
        pl.kernel,
        out_type=jax.ShapeDtypeStruct((_NC, _NPAD, 2 * H), _f32),
        mesh=mesh,
        scratch_types=[
            pltpu.VMEM((_K, _C), jnp.int32),
            pltpu.VMEM((_K, _C), jnp.int32),
            pltpu.VMEM((3, _C, H), _f32),
            pltpu.VMEM((3, _C, H), _f32),
            pltpu.VMEM((2, _C, 2 * H), _f32),
            pltpu.VMEM((2 * H,), _f32),
            pltpu.VMEM_SHARED((_NPAD, 2 * H), _f32),
            pltpu.SemaphoreType.DMA,
            pltpu.SemaphoreType.DMA,
            pltpu.SemaphoreType.DMA,
            pltpu.SemaphoreType.DMA,
            pltpu.SemaphoreType.DMA,
            pltpu.SemaphoreType.DMA,
            pltpu.SemaphoreType.DMA,
            pltpu.SemaphoreType.DMA,
        ],
        compiler_params=pltpu.CompilerParams(use_tc_tiling_on_sc=False),
    )
    def kern(h_hbm, ea_hbm, src_hbm, dst_hbm, par_hbm, z_hbm, out_hbm,
             sidx, didx, gbuf, ebuf, obuf, pbuf, acc,
             semg0, semg1, semg2, seme0, seme1, seme2, sems0, sems1):
        cid = lax.axis_index("c")
        sid = lax.axis_index("s")
        wid = cid * _NS + sid
        semg = (semg0, semg1, semg2)
        seme = (seme0, seme1, seme2)
        sems = (sems0, sems1)

        pltpu.sync_copy(par_hbm, pbuf)
        pltpu.sync_copy(src_hbm.at[pl.ds(wid * _K, _K)], sidx)
        pltpu.sync_copy(dst_hbm.at[pl.ds(wid * _K, _K)], didx)
        pltpu.sync_copy(z_hbm, acc.at[pl.ds(sid * _RPT, _RPT)])
        plsc.subcore_barrier()

        tv = pbuf[pl.ds(H, 16)]
        dvs = tuple(pbuf[pl.ds(16 * q, 16)] for q in range(4))

        def fire(k, b):
            pltpu.async_copy(h_hbm.at[sidx.at[k]], gbuf.at[b], semg[b])
            base = wid * _EW + k * _C
            pltpu.async_copy(ea_hbm.at[pl.ds(base, _C)], ebuf.at[b], seme[b])

        def wait_in(b):
            pltpu.make_async_copy(h_hbm.at[sidx.at[0]], gbuf.at[b], semg[b]).wait()
            pltpu.make_async_copy(ea_hbm.at[pl.ds(0, _C)], ebuf.at[b], seme[b]).wait()

        def compute(b, o):
            @plsc.parallel_loop(0, _C, 1, unroll=4)
            def row(i):
                gs = [gbuf[b, i, pl.ds(16 * q, 16)] for q in range(4)]
                es = [ebuf[b, i, pl.ds(16 * q, 16)] for q in range(4)]
                vs = [jnp.maximum(gs[q] + es[q], 0.0) for q in range(4)]
                ws = [jnp.exp(tv * vs[q] + dvs[q]) for q in range(4)]
                for q in range(4):
                    obuf[o, i, pl.ds(16 * q, 16)] = (vs[q] + EPS) * ws[q]
                    obuf[o, i, pl.ds(H + 16 * q, 16)] = ws[q]

        def scat(k, o):
            pltpu.async_copy(obuf.at[o], acc.at[didx.at[k]], sems[o], add=True)

        def wait_scat(o):
            pltpu.make_async_copy(obuf.at[o], acc.at[didx.at[0]], sems[o]).wait()

        fire(0, 0)
        fire(1, 1)

        def group(g, carry):
            k0 = 6 * g
            for u in range(6):
                k = k0 + u
                fire(k + 2, (u + 2) % 3)
                wait_in(u % 3)
                if u < 2:
                    @pl.when(g >= 1)
                    def _(u=u):
                        wait_scat(u % 2)
                else:
                    wait_scat(u % 2)
                compute(u % 3, u % 2)
                scat(k, u % 2)
            return carry

        lax.fori_loop(0, (_K - 4) // 6, group, 0)

        for k in range(_K - 4, _K):
            u3 = k % 3
            u2 = k % 2
            if k + 2 < _K:
                fire(k + 2, (k + 2) % 3)
            wait_in(u3)
            wait_scat(u2)
            compute(u3, u2)
            scat(k, u2)

        wait_scat(0)
        wait_scat(1)

        plsc.subcore_barrier()
        pltpu.sync_copy(acc.at[pl.ds(sid * _RPT, _RPT)],
                        out_hbm.at[cid, pl.ds(sid * _RPT, _RPT)])

    return kern(h, ea, srcr, dstr, par, zrows)


def kernel(x, edge_index, edge_attr, batch, node_W, node_b, e1_W, e1_b, e2_W,
           e2_b, ln_g, ln_b, mlp1_W, mlp1_b, mln_g, mln_b, mlp2_W, mlp2_b, t,
           out_W, out_b):
    srcr = edge_index[0].reshape(E // _C, _C)
    dstr = edge_index[1].reshape(E // _C, _C)

    h0, hmax8 = _node_enc(x, node_W, node_b.reshape(1, H))
    ea, emax8 = _edge_mlp(edge_attr, e1_W, e1_b.reshape(1, 32), e2_W,
                          e2_b.reshape(1, H))
    eamax = emax8[0]

    zrows = jnp.zeros((_RPT, 2 * H), _f32)
    res = jnp.zeros((N, H), _f32)
    hin = h0
    hmaxv = hmax8[0]

    for i in range(L):
        ti = t[i]
        c = jnp.maximum(ti * (jnp.maximum(hmaxv + eamax, 0.0) + EPS), ti * EPS)
        par = jnp.concatenate([ti * EPS - c, jnp.full((H,), 1.0, _f32) * ti])
        parts = _edge_pass(hin, ea, srcr, dstr, par, zrows)
        if i < L - 1:
            lng, lnb = ln_g[i + 1], ln_b[i + 1]
        else:
            lng, lnb = ln_g[0], ln_b[0]
        h_new, hn, hmax8 = _combine(
            parts, hin, res,
            mlp1_W[i], mlp1_b[i].reshape(1, 2 * H),
            mln_g[i].reshape(1, 2 * H), mln_b[i].reshape(1, 2 * H),
            mlp2_W[i], mlp2_b[i].reshape(1, H),
            lng.reshape(1, H), lnb.reshape(1, H))
        res = h_new
        hin = hn
        hmaxv = hmax8[0]

    pooled = hmaxv.reshape(1, H)
    return jax.nn.sigmoid(pooled @ out_W + out_b)


# speedup vs baseline: 1.1997x; 1.0986x over previous
import functools

import jax
import jax.numpy as jnp
from jax import lax
from jax.experimental import pallas as pl
from jax.experimental.pallas import tpu as pltpu
from jax.experimental.pallas import tpu_sc as plsc

N = 10000
E = 320000
D_FEAT = 128
D_EDGE = 16
H = 64
L = 3
EPS = 1e-7

_NC = 2
_NS = 16
_NW = _NC * _NS
_EW = E // _NW
_C = 40
_K = _EW // _C
_NPAD = 10240
_RPT = _NPAD // _NS

_f32 = jnp.float32
_i32 = jnp.int32


def _pack_bf16(m):
    lo = jax.lax.bitcast_convert_type(
        m[:, :H // 2].astype(jnp.bfloat16).astype(_f32), _i32)
    hi = jax.lax.bitcast_convert_type(
        m[:, H // 2:].astype(jnp.bfloat16).astype(_f32), _i32)
    lo16 = jnp.bitwise_and(jax.lax.shift_right_logical(lo, jnp.full_like(lo, 16)),
                           jnp.full_like(lo, 0xFFFF))
    hi16 = jnp.bitwise_and(hi, jnp.full_like(hi, -65536))
    return jnp.bitwise_or(lo16, hi16)


def _node_enc(x, w, b):
    R = 2000

    def body(x_ref, w_ref, b_ref, h_ref, hp_ref, hmax_ref):
        h = jnp.dot(x_ref[...], w_ref[...], preferred_element_type=_f32) + b_ref[...]
        h_ref[...] = h
        hp_ref[...] = _pack_bf16(h)
        m8 = jnp.broadcast_to(jnp.max(h, axis=0, keepdims=True), (8, H))

        @pl.when(pl.program_id(0) == 0)
        def _():
            hmax_ref[...] = m8

        @pl.when(pl.program_id(0) > 0)
        def _():
            hmax_ref[...] = jnp.maximum(hmax_ref[...], m8)

    return pl.pallas_call(
        body,
        grid=(N // R,),
        in_specs=[
            pl.BlockSpec((R, D_FEAT), lambda i: (i, 0)),
            pl.BlockSpec((D_FEAT, H), lambda i: (0, 0)),
            pl.BlockSpec((1, H), lambda i: (0, 0)),
        ],
        out_specs=[
            pl.BlockSpec((R, H), lambda i: (i, 0)),
            pl.BlockSpec((R, H // 2), lambda i: (i, 0)),
            pl.BlockSpec((8, H), lambda i: (0, 0)),
        ],
        out_shape=[
            jax.ShapeDtypeStruct((N, H), _f32),
            jax.ShapeDtypeStruct((N, H // 2), _i32),
            jax.ShapeDtypeStruct((8, H), _f32),
        ],
    )(x, w, b)


def _edge_mlp(a, w1, b1, w2, b2):
    R = 8000

    def body(a_ref, w1_ref, b1_ref, w2_ref, b2_ref, ea_ref, emax_ref):
        u = jnp.maximum(
            jnp.dot(a_ref[...], w1_ref[...], preferred_element_type=_f32) + b1_ref[...], 0.0)
        ea = jnp.dot(u, w2_ref[...], preferred_element_type=_f32) + b2_ref[...]
        ea_ref[...] = _pack_bf16(ea)
        m8 = jnp.broadcast_to(jnp.max(ea, axis=0, keepdims=True), (8, H))

        @pl.when(pl.program_id(0) == 0)
        def _():
            emax_ref[...] = m8

        @pl.when(pl.program_id(0) > 0)
        def _():
            emax_ref[...] = jnp.maximum(emax_ref[...], m8)

    return pl.pallas_call(
        body,
        grid=(E // R,),
        in_specs=[
            pl.BlockSpec((R, D_EDGE), lambda i: (i, 0)),
            pl.BlockSpec((D_EDGE, 32), lambda i: (0, 0)),
            pl.BlockSpec((1, 32), lambda i: (0, 0)),
            pl.BlockSpec((32, H), lambda i: (0, 0)),
            pl.BlockSpec((1, H), lambda i: (0, 0)),
        ],
        out_specs=[
            pl.BlockSpec((R, H // 2), lambda i: (i, 0)),
            pl.BlockSpec((8, H), lambda i: (0, 0)),
        ],
        out_shape=[
            jax.ShapeDtypeStruct((E, H // 2), _i32),
            jax.ShapeDtypeStruct((8, H), _f32),
        ],
    )(a, w1, b1, w2, b2)


def _combine(parts, hin, res, w1, b1, g1, bb1, w2, b2, lng, lnb):
    R = 2000

    def body(p0_ref, p1_ref, hin_ref, res_ref, w1_ref, b1_ref, g1_ref, bb1_ref,
             w2_ref, b2_ref, lng_ref, lnb_ref, h_ref, hn_ref, hnp_ref, hmax_ref):
        p0 = p0_ref[0]
        p1 = p1_ref[0]
        num = p0[:, :H] + p1[:, :H]
        den = p0[:, H:] + p1[:, H:]
        aggr = num / (den + 1e-16)
        v = aggr + hin_ref[...]
        u = jnp.dot(v, w1_ref[...], preferred_element_type=_f32) + b1_ref[...]
        mu = jnp.mean(u, axis=-1, keepdims=True)
        var = jnp.mean((u - mu) ** 2, axis=-1, keepdims=True)
        u = (u - mu) / jnp.sqrt(var + 1e-5) * g1_ref[...] + bb1_ref[...]
        u = jnp.maximum(u, 0.0)
        h = res_ref[...] + jnp.dot(u, w2_ref[...], preferred_element_type=_f32) + b2_ref[...]
        h_ref[...] = h
        mu2 = jnp.mean(h, axis=-1, keepdims=True)
        var2 = jnp.mean((h - mu2) ** 2, axis=-1, keepdims=True)
        hn = jnp.maximum((h - mu2) / jnp.sqrt(var2 + 1e-5) * lng_ref[...] + lnb_ref[...], 0.0)
        hn_ref[...] = hn
        hnp_ref[...] = _pack_bf16(hn)
        m8 = jnp.broadcast_to(jnp.max(hn, axis=0, keepdims=True), (8, H))

        @pl.when(pl.program_id(0) == 0)
        def _():
            hmax_ref[...] = m8

        @pl.when(pl.program_id(0) > 0)
        def _():
            hmax_ref[...] = jnp.maximum(hmax_ref[...], m8)

    nblk = N // R
    return pl.pallas_call(
        body,
        grid=(nblk,),
        in_specs=[
            pl.BlockSpec((1, R, 2 * H), lambda i: (0, i, 0)),
            pl.BlockSpec((1, R, 2 * H), lambda i: (1, i, 0)),
            pl.BlockSpec((R, H), lambda i: (i, 0)),
            pl.BlockSpec((R, H), lambda i: (i, 0)),
            pl.BlockSpec((H, 2 * H), lambda i: (0, 0)),
            pl.BlockSpec((1, 2 * H), lambda i: (0, 0)),
            pl.BlockSpec((1, 2 * H), lambda i: (0, 0)),
            pl.BlockSpec((1, 2 * H), lambda i: (0, 0)),
            pl.BlockSpec((2 * H, H), lambda i: (0, 0)),
            pl.BlockSpec((1, H), lambda i: (0, 0)),
            pl.BlockSpec((1, H), lambda i: (0, 0)),
            pl.BlockSpec((1, H), lambda i: (0, 0)),
        ],
        out_specs=[
            pl.BlockSpec((R, H), lambda i: (i, 0)),
            pl.BlockSpec((R, H), lambda i: (i, 0)),
            pl.BlockSpec((R, H // 2), lambda i: (i, 0)),
            pl.BlockSpec((8, H), lambda i: (0, 0)),
        ],
        out_shape=[
            jax.ShapeDtypeStruct((N, H), _f32),
            jax.ShapeDtypeStruct((N, H), _f32),
            jax.ShapeDtypeStruct((N, H // 2), _i32),
            jax.ShapeDtypeStruct((8, H), _f32),
        ],
    )(parts, parts, hin, res, w1, b1, g1, bb1, w2, b2, lng, lnb)


def _edge_pass(h, ea, srcr, dstr, par, zrows):
    mesh = plsc.VectorSubcoreMesh(core_axis_name="c", subcore_axis_name="s")

    @functools.partial(
        pl.kernel,
        out_type=jax.ShapeDtypeStruct((_NC, _NPAD, 2 * H), _f32),
        mesh=mesh,
        scratch_types=[
            pltpu.VMEM((_K, _C), jnp.int32),
            pltpu.VMEM((_K, _C), jnp.int32),
            pltpu.VMEM((3, _C, H // 2), _i32),
            pltpu.VMEM((3, _C, H // 2), _i32),
            pltpu.VMEM((2, _C, 2 * H), _f32),
            pltpu.VMEM((2 * H,), _f32),
            pltpu.VMEM_SHARED((_NPAD, 2 * H), _f32),
            pltpu.SemaphoreType.DMA,
            pltpu.SemaphoreType.DMA,
            pltpu.SemaphoreType.DMA,
            pltpu.SemaphoreType.DMA,
            pltpu.SemaphoreType.DMA,
            pltpu.SemaphoreType.DMA,
            pltpu.SemaphoreType.DMA,
            pltpu.SemaphoreType.DMA,
        ],
        compiler_params=pltpu.CompilerParams(use_tc_tiling_on_sc=False,
                                             needs_layout_passes=False),
    )
    def kern(h_hbm, ea_hbm, src_hbm, dst_hbm, par_hbm, z_hbm, out_hbm,
             sidx, didx, gbuf, ebuf, obuf, pbuf, acc,
             semg0, semg1, semg2, seme0, seme1, seme2, sems0, sems1):
        cid = lax.axis_index("c")
        sid = lax.axis_index("s")
        wid = cid * _NS + sid
        semg = (semg0, semg1, semg2)
        seme = (seme0, seme1, seme2)
        sems = (sems0, sems1)

        pltpu.sync_copy(par_hbm, pbuf)
        pltpu.sync_copy(src_hbm.at[pl.ds(wid * _K, _K)], sidx)
        pltpu.sync_copy(dst_hbm.at[pl.ds(wid * _K, _K)], didx)
        pltpu.sync_copy(z_hbm, acc.at[pl.ds(sid * _RPT, _RPT)])
        plsc.subcore_barrier()

        tv = pbuf[pl.ds(H, 16)]
        dvs = tuple(pbuf[pl.ds(16 * q, 16)] for q in range(4))

        def fire(k, b):
            pltpu.async_copy(h_hbm.at[sidx.at[k]], gbuf.at[b], semg[b])
            base = wid * _EW + k * _C
            pltpu.async_copy(ea_hbm.at[pl.ds(base, _C)], ebuf.at[b], seme[b])

        def wait_in(b):
            pltpu.make_async_copy(h_hbm.at[sidx.at[0]], gbuf.at[b], semg[b]).wait()
            pltpu.make_async_copy(ea_hbm.at[pl.ds(0, _C)], ebuf.at[b], seme[b]).wait()

        def compute(b, o):
            sixteen = jnp.full((16,), 16, _i32)

            @plsc.parallel_loop(0, _C, 1, unroll=4)
            def row(i):
                gw = [gbuf[b, i, pl.ds(16 * j, 16)] for j in range(2)]
                ew = [ebuf[b, i, pl.ds(16 * j, 16)] for j in range(2)]
                gs = [plsc.bitcast(jax.lax.shift_left(gw[0], sixteen), _f32),
                      plsc.bitcast(jax.lax.shift_left(gw[1], sixteen), _f32),
                      plsc.bitcast(gw[0], _f32),
                      plsc.bitcast(gw[1], _f32)]
                es = [plsc.bitcast(jax.lax.shift_left(ew[0], sixteen), _f32),
                      plsc.bitcast(jax.lax.shift_left(ew[1], sixteen), _f32),
                      plsc.bitcast(ew[0], _f32),
                      plsc.bitcast(ew[1], _f32)]
                vs = [jnp.maximum(gs[q] + es[q], 0.0) for q in range(4)]
                ws = [jnp.exp(tv * vs[q] + dvs[q]) for q in range(4)]
                for q in range(4):
                    obuf[o, i, pl.ds(16 * q, 16)] = (vs[q] + EPS) * ws[q]
                    obuf[o, i, pl.ds(H + 16 * q, 16)] = ws[q]

        def scat(k, o):
            pltpu.async_copy(obuf.at[o], acc.at[didx.at[k]], sems[o], add=True)

        def wait_scat(o):
            pltpu.make_async_copy(obuf.at[o], acc.at[didx.at[0]], sems[o]).wait()

        fire(0, 0)
        fire(1, 1)

        def group(g, carry):
            k0 = 6 * g
            for u in range(6):
                k = k0 + u
                fire(k + 2, (u + 2) % 3)
                wait_in(u % 3)
                if u < 2:
                    @pl.when(g >= 1)
                    def _(u=u):
                        wait_scat(u % 2)
                else:
                    wait_scat(u % 2)
                compute(u % 3, u % 2)
                scat(k, u % 2)
            return carry

        lax.fori_loop(0, (_K - 4) // 6, group, 0)

        for k in range(_K - 4, _K):
            u3 = k % 3
            u2 = k % 2
            if k + 2 < _K:
                fire(k + 2, (k + 2) % 3)
            wait_in(u3)
            wait_scat(u2)
            compute(u3, u2)
            scat(k, u2)

        wait_scat(0)
        wait_scat(1)

        plsc.subcore_barrier()
        pltpu.sync_copy(acc.at[pl.ds(sid * _RPT, _RPT)],
                        out_hbm.at[cid, pl.ds(sid * _RPT, _RPT)])

    return kern(h, ea, srcr, dstr, par, zrows)


def kernel(x, edge_index, edge_attr, batch, node_W, node_b, e1_W, e1_b, e2_W,
           e2_b, ln_g, ln_b, mlp1_W, mlp1_b, mln_g, mln_b, mlp2_W, mlp2_b, t,
           out_W, out_b):
    srcr = edge_index[0].reshape(E // _C, _C)
    dstr = edge_index[1].reshape(E // _C, _C)

    h0, hp0, hmax8 = _node_enc(x, node_W, node_b.reshape(1, H))
    eap, emax8 = _edge_mlp(edge_attr, e1_W, e1_b.reshape(1, 32), e2_W,
                           e2_b.reshape(1, H))
    eamax = emax8[0]

    zrows = jnp.zeros((_RPT, 2 * H), _f32)
    res = jnp.zeros((N, H), _f32)
    hin = h0
    hp = hp0
    hmaxv = hmax8[0]

    for i in range(L):
        ti = t[i]
        c = jnp.maximum(ti * (jnp.maximum(hmaxv + eamax, 0.0) + EPS), ti * EPS)
        par = jnp.concatenate([ti * EPS - c, jnp.full((H,), 1.0, _f32) * ti])
        parts = _edge_pass(hp, eap, srcr, dstr, par, zrows)
        if i < L - 1:
            lng, lnb = ln_g[i + 1], ln_b[i + 1]
        else:
            lng, lnb = ln_g[0], ln_b[0]
        h_new, hn, hnp, hmax8 = _combine(
            parts, hin, res,
            mlp1_W[i], mlp1_b[i].reshape(1, 2 * H),
            mln_g[i].reshape(1, 2 * H), mln_b[i].reshape(1, 2 * H),
            mlp2_W[i], mlp2_b[i].reshape(1, H),
            lng.reshape(1, H), lnb.reshape(1, H))
        res = h_new
        hin = hn
        hp = hnp
        hmaxv = hmax8[0]

    pooled = hmaxv.reshape(1, H)
    return jax.nn.sigmoid(pooled @ out_W + out_b)
